# Initial kernel scaffold; baseline (speedup 1.0000x reference)
import jax
import jax.numpy as jnp
from jax.experimental import pallas as pl


def kernel(x1, edge_index1, x2, edge_index2, W1, b1, W2, b2):
    raise NotImplementedError("write your pallas kernel here")


# speedup vs baseline: 7.2534x; 7.2534x over previous
import functools

import jax
import jax.numpy as jnp
from jax import lax
from jax.experimental import pallas as pl
from jax.experimental.pallas import tpu as pltpu
from jax.experimental.pallas import tpu_sc as plsc

_N = 10000
_E = 160000
_D = 256
_DH = 128
_NSUB = 16
_NCORE = 2
_RPT = 624
_RPT_LAST = _N - (_NSUB - 1) * _RPT
_EDGES_PER_TILE = _E // _NSUB
_B = 80
_NB = _EDGES_PER_TILE // _B
_EDGES_PER_TILE32 = _E // (_NSUB * _NCORE)
_BR = 1000
_RB = _N // _BR

_sc_mesh = plsc.VectorSubcoreMesh(core_axis_name="c", subcore_axis_name="s")


@functools.partial(
    pl.kernel,
    out_type=jax.ShapeDtypeStruct((_NSUB * _NCORE, _N), jnp.float32),
    mesh=_sc_mesh,
    scratch_types=[
        pltpu.VMEM((_N,), jnp.float32),
        pltpu.VMEM((_EDGES_PER_TILE32 + 16,), jnp.int32),
    ],
    compiler_params=pltpu.CompilerParams(needs_layout_passes=False),
)
def _deg_sc(dst_hbm, out_hbm, cnt_v, buf_v):
    c = lax.axis_index("c")
    s = lax.axis_index("s")
    wid = s * _NCORE + c

    def zero_body(i, carry):
        cnt_v[pl.ds(i * 16, 16)] = jnp.zeros((16,), jnp.float32)
        return carry

    lax.fori_loop(0, _N // 16, zero_body, 0)

    pltpu.sync_copy(
        dst_hbm.at[pl.ds(wid * _EDGES_PER_TILE32, _EDGES_PER_TILE32)],
        buf_v.at[pl.ds(0, _EDGES_PER_TILE32)],
    )

    ones = jnp.ones((16,), jnp.float32)
    full = _EDGES_PER_TILE32 // 16
    rem = _EDGES_PER_TILE32 - full * 16

    def body(i, carry):
        idx = buf_v[pl.ds(i * 16, 16)]
        plsc.addupdate_scatter(cnt_v, [idx], ones)
        return carry

    lax.fori_loop(0, full, body, 0)
    if rem:
        lanes = lax.iota(jnp.int32, 16)
        m = lanes < rem
        tail = buf_v[pl.ds(full * 16, 16)]
        tail = jnp.where(m, tail, 0)
        plsc.addupdate_scatter(cnt_v, [tail], ones, mask=m)

    pltpu.sync_copy(cnt_v, out_hbm.at[wid])


@functools.partial(
    pl.kernel,
    out_type=jax.ShapeDtypeStruct((_NCORE * _N, _DH), jnp.float32),
    mesh=_sc_mesh,
    scratch_types=[
        pltpu.VMEM((_B,), jnp.int32),
        pltpu.VMEM((_B,), jnp.int32),
        pltpu.VMEM((_B,), jnp.int32),
        pltpu.VMEM((_B, _DH), jnp.float32),
        pltpu.VMEM_SHARED((_N, _DH), jnp.float32),
        pltpu.SemaphoreType.DMA,
    ],
    compiler_params=pltpu.CompilerParams(needs_layout_passes=False),
)
def _segsum_sc(y_hbm, src_hbm, dst_hbm, zeros_hbm, out_hbm,
               src_v, dst_v, gidx_v, rows_v, acc_sh, sem):
    c = lax.axis_index("c")
    s = lax.axis_index("s")

    last = _NSUB - 1

    @pl.when(s < last)
    def _():
        pltpu.sync_copy(zeros_hbm.at[pl.ds(0, _RPT)],
                        acc_sh.at[pl.ds(s * _RPT, _RPT)])

    @pl.when(s == last)
    def _():
        pltpu.sync_copy(zeros_hbm, acc_sh.at[pl.ds(last * _RPT, _RPT_LAST)])

    plsc.subcore_barrier()

    off = c * _N

    def body(i, carry):
        base = s * _EDGES_PER_TILE + i * _B
        pltpu.sync_copy(src_hbm.at[pl.ds(base, _B)], src_v)
        pltpu.sync_copy(dst_hbm.at[pl.ds(base, _B)], dst_v)
        for k in range(_B // 16):
            gidx_v[pl.ds(k * 16, 16)] = src_v[pl.ds(k * 16, 16)] + off
        pltpu.async_copy(y_hbm.at[gidx_v], rows_v, sem).wait()
        pltpu.sync_copy(rows_v, acc_sh.at[dst_v], add=True)
        return carry

    lax.fori_loop(0, _NB, body, 0)
    plsc.subcore_barrier()

    @pl.when(s < last)
    def _():
        pltpu.sync_copy(acc_sh.at[pl.ds(s * _RPT, _RPT)],
                        out_hbm.at[pl.ds(off + s * _RPT, _RPT)])

    @pl.when(s == last)
    def _():
        pltpu.sync_copy(acc_sh.at[pl.ds(last * _RPT, _RPT_LAST)],
                        out_hbm.at[pl.ds(off + last * _RPT, _RPT_LAST)])


def _dinv_body(p_ref, o_ref):
    deg = 1.0 + jnp.sum(p_ref[...], axis=0)
    o_ref[...] = lax.rsqrt(deg)[:, None]


def _dinv_tc(parts):
    return pl.pallas_call(
        _dinv_body,
        out_shape=jax.ShapeDtypeStruct((_N, 1), jnp.float32),
    )(parts)


def _mm1_body(x_ref, w_ref, dinv_ref, y_ref):
    y_ref[...] = jnp.dot(x_ref[...], w_ref[...],
                         preferred_element_type=jnp.float32) * dinv_ref[...]


def _mm1(x, W, dinv):
    return pl.pallas_call(
        _mm1_body,
        grid=(_RB, _NCORE),
        in_specs=[
            pl.BlockSpec((_BR, _D), lambda b, c: (b, 0)),
            pl.BlockSpec((_D, _DH), lambda b, c: (0, c)),
            pl.BlockSpec((_BR, 1), lambda b, c: (b, 0)),
        ],
        out_specs=pl.BlockSpec((_BR, _DH), lambda b, c: (c * _RB + b, 0)),
        out_shape=jax.ShapeDtypeStruct((_NCORE * _N, _DH), jnp.float32),
    )(x, W, dinv)


def _mm2_body(acc_lo_ref, acc_hi_ref, y_lo_ref, y_hi_ref, dinv_ref, b_ref,
              w_ref, out_ref):
    dinv = dinv_ref[...]
    lo = dinv * (acc_lo_ref[...] + y_lo_ref[...])
    hi = dinv * (acc_hi_ref[...] + y_hi_ref[...])
    h = jnp.concatenate([lo, hi], axis=1) + b_ref[...][None, :]
    h = jnp.maximum(h, 0.0)
    out_ref[...] = jnp.dot(h, w_ref[...],
                           preferred_element_type=jnp.float32) * dinv


def _mm2(acc, y, dinv, bvec, W):
    return pl.pallas_call(
        _mm2_body,
        grid=(_RB, _NCORE),
        in_specs=[
            pl.BlockSpec((_BR, _DH), lambda b, c: (b, 0)),
            pl.BlockSpec((_BR, _DH), lambda b, c: (_RB + b, 0)),
            pl.BlockSpec((_BR, _DH), lambda b, c: (b, 0)),
            pl.BlockSpec((_BR, _DH), lambda b, c: (_RB + b, 0)),
            pl.BlockSpec((_BR, 1), lambda b, c: (b, 0)),
            pl.BlockSpec((_D,), lambda b, c: (0,)),
            pl.BlockSpec((_D, _DH), lambda b, c: (0, c)),
        ],
        out_specs=pl.BlockSpec((_BR, _DH), lambda b, c: (c * _RB + b, 0)),
        out_shape=jax.ShapeDtypeStruct((_NCORE * _N, _DH), jnp.float32),
    )(acc, acc, y, y, dinv, bvec, W)


def _final_body(acc_ref, y_ref, dinv_ref, b_ref, z_ref):
    dinv = dinv_ref[...]
    a = acc_ref[...]
    yv = y_ref[...]
    lo = dinv * (a[:_N] + yv[:_N])
    hi = dinv * (a[_N:] + yv[_N:])
    out = jnp.concatenate([lo, hi], axis=1) + b_ref[...][None, :]
    mu = jnp.mean(out, axis=0)
    d = out - mu[None, :]
    var = jnp.sum(d * d, axis=0) / (_N - 1)
    z_ref[...] = d * lax.rsqrt(var)[None, :]


def _final(acc, y, dinv, bvec):
    return pl.pallas_call(
        _final_body,
        out_shape=jax.ShapeDtypeStruct((_N, _D), jnp.float32),
    )(acc, y, dinv, bvec)


def kernel(x1, edge_index1, x2, edge_index2, W1, b1, W2, b2):
    zeros = jnp.zeros((_RPT_LAST, _DH), jnp.float32)

    def backbone(x, ei):
        src, dst = ei[0], ei[1]
        parts = _deg_sc(dst)
        dinv = _dinv_tc(parts)
        y1 = _mm1(x, W1, dinv)
        acc1 = _segsum_sc(y1, src, dst, zeros)
        y2 = _mm2(acc1, y1, dinv, b1, W2)
        acc2 = _segsum_sc(y2, src, dst, zeros)
        return _final(acc2, y2, dinv, b2)

    return backbone(x1, edge_index1), backbone(x2, edge_index2)
